# Initial kernel scaffold; baseline (speedup 1.0000x reference)
import jax
import jax.numpy as jnp
from jax.experimental import pallas as pl


def kernel(indices, table):
    raise NotImplementedError("write your pallas kernel here")


# speedup vs baseline: 4.3894x; 4.3894x over previous
import functools

import jax
import jax.numpy as jnp
from jax import lax
from jax.experimental import pallas as pl
from jax.experimental.pallas import tpu as pltpu
from jax.experimental.pallas import tpu_sc as plsc

NUM_POS = 17
EMB = 10
ROWS = 16384
SEQ = 200
TOK = ROWS * SEQ

NC = 2
NS = 16
NW = NC * NS
TOK_PER_W = TOK // NW
CHUNK = 4096
ROUNDS = TOK_PER_W // CHUNK
GROUPS = CHUNK // 16

_mesh = plsc.VectorSubcoreMesh(core_axis_name="c", subcore_axis_name="s",
                               num_cores=NC, num_subcores=NS)


@functools.partial(
    pl.kernel,
    out_type=jax.ShapeDtypeStruct((TOK * EMB,), jnp.float32),
    mesh=_mesh,
    scratch_types=[
        pltpu.VMEM((NUM_POS * EMB,), jnp.float32),
        pltpu.VMEM((CHUNK,), jnp.int32),
        pltpu.VMEM((CHUNK * EMB,), jnp.float32),
    ],
    compiler_params=pltpu.CompilerParams(needs_layout_passes=False),
)
def _emb_kernel(idx_hbm, table_hbm, out_hbm, table_v, idx_v, out_v):
    wid = lax.axis_index("s") * NC + lax.axis_index("c")
    base = wid * TOK_PER_W

    pltpu.sync_copy(table_hbm, table_v)

    lane10 = lax.iota(jnp.int32, 16) * EMB

    def round_body(r, _):
        tok0 = base + r * CHUNK
        pltpu.sync_copy(idx_hbm.at[pl.ds(tok0, CHUNK)], idx_v)

        def group_body(g, _):
            g16 = g * 16
            rows10 = idx_v[pl.ds(g16, 16)] * EMB
            dst10 = g16 * EMB + lane10
            for d in range(EMB):
                vals = plsc.load_gather(table_v, [rows10 + d])
                plsc.store_scatter(out_v, [dst10 + d], vals)
            return 0

        lax.fori_loop(0, GROUPS, group_body, 0, unroll=False)
        pltpu.sync_copy(out_v, out_hbm.at[pl.ds(tok0 * EMB, CHUNK * EMB)])
        return 0

    lax.fori_loop(0, ROUNDS, round_body, 0, unroll=False)


def kernel(indices, table):
    idx_flat = indices.reshape(TOK).astype(jnp.int32)
    out = _emb_kernel(idx_flat, table.reshape(NUM_POS * EMB))
    return out.reshape(ROWS, SEQ, EMB)
